# baseline (speedup 1.0000x reference)
import functools

import jax
import jax.numpy as jnp
from jax import lax
from jax.experimental import pallas as pl
from jax.experimental.pallas import tpu as pltpu
from jax.experimental.pallas import tpu_sc as plsc

NC = 2
NS = 16
L = 16

F = 9
V = 1000

NB = 32
WB = NB * F
FRAME = F * L
VPB = WB // L
GROUPS = 3
GROW = WB // GROUPS
SPMEM_GROUPS = 2

INT_MAX = 2147483647


def _adjust_body(x_ref, adj_ref):
  m = jnp.min(x_ref[...], axis=0, keepdims=True)
  feat = lax.broadcasted_iota(jnp.int32, (1, FRAME), 1) % F
  adj = jnp.zeros((1, FRAME), jnp.int32)
  for fi in range(F):
    mask = feat == fi
    mfi = jnp.min(jnp.where(mask, m, INT_MAX), axis=1, keepdims=True)
    adj = jnp.where(mask, V * fi - mfi, adj)
  adj_ref[...] = adj


def _sc_body(NBLK, KMAX, xflat, adj_hbm, comb, out,
             xbuf0, xbuf1, adjbuf, idxbuf0, idxbuf1,
             gbuf0, gbuf1, obuf0, obuf1, stab,
             gsem0, gsem1, hsem0, hsem1, osem0, osem1):
  c = lax.axis_index("c")
  s = lax.axis_index("s")
  wid = s * NC + c

  xbufs = (xbuf0, xbuf1)
  idxbufs = (idxbuf0, idxbuf1)
  gbufs = (gbuf0, gbuf1)
  obufs = (obuf0, obuf1)
  gsems = (gsem0, gsem1)
  hsems = (hsem0, hsem1)
  osems = (osem0, osem1)

  @pl.when(s == 0)
  def _():
    pltpu.sync_copy(comb, stab)

  plsc.subcore_barrier()

  pltpu.sync_copy(adj_hbm, adjbuf)

  SROWS = SPMEM_GROUPS * GROW

  def prep(j, p):
    xbuf, idxbuf = xbufs[p], idxbufs[p]
    pltpu.sync_copy(xflat.at[pl.ds(j * WB, WB)], xbuf)
    for v in range(VPB):
      idx = xbuf[pl.ds(v * L, L)] + adjbuf[0, pl.ds((v % F) * L, L)]
      idxbuf[v // (GROW // L), pl.ds((v % (GROW // L)) * L, L)] = idx
    for g in range(GROUPS):
      if g < SPMEM_GROUPS:
        pltpu.async_copy(stab.at[idxbuf.at[g]],
                         gbufs[p].at[pl.ds(g * GROW, GROW)], gsems[p])
      else:
        pltpu.async_copy(comb.at[idxbuf.at[g]],
                         gbufs[p].at[pl.ds(g * GROW, GROW)], hsems[p])

  def wait_gathers(p):
    pltpu.make_async_copy(stab.at[pl.ds(0, SROWS)],
                          gbufs[p].at[pl.ds(0, SROWS)], gsems[p]).wait()
    pltpu.make_async_copy(comb.at[pl.ds(0, WB - SROWS)],
                          gbufs[p].at[pl.ds(SROWS, WB - SROWS)],
                          hsems[p]).wait()

  def accum_and_out(j, p):
    gbuf, obuf = gbufs[p], obufs[p]

    def node_body(n, _):
      for cc in range(4):
        acc = plsc.bitcast(gbuf[n * F, pl.ds(cc * L, L)], jnp.bfloat16)
        for fi in range(1, F):
          acc = acc + plsc.bitcast(gbuf[n * F + fi, pl.ds(cc * L, L)],
                                   jnp.bfloat16)
        obuf[n, pl.ds(cc * L, L)] = plsc.bitcast(acc, jnp.int32)
      return 0

    lax.fori_loop(0, NB, node_body, 0)
    pltpu.async_copy(obuf, out.at[pl.ds(j * NB, NB)], osems[p])

  def step(k, p):
    jn = wid + 32 * (k + 1)

    @pl.when(jn < NBLK)
    def _():
      prep(jn, 1 - p)

    jc = wid + 32 * k

    @pl.when(jc < NBLK)
    def _():
      wait_gathers(p)

      @pl.when(k >= 2)
      def _():
        pltpu.make_async_copy(obufs[p],
                              out.at[pl.ds((jc - 64) * NB, NB)],
                              osems[p]).wait()

      accum_and_out(jc, p)

  prep(wid, 0)

  def pair_body(m, _):
    step(2 * m, 0)
    step(2 * m + 1, 1)
    return 0

  lax.fori_loop(0, KMAX // 2, pair_body, 0)

  kstar = (NBLK - 1 - wid) // 32
  for p in (0, 1):
    kp_last = kstar - ((kstar - p) % 2)

    @pl.when(kp_last >= 0)
    def _():
      pltpu.make_async_copy(obufs[p],
                            out.at[pl.ds((wid + 32 * kp_last) * NB, NB)],
                            osems[p]).wait()


@functools.partial(jax.jit, static_argnums=(2, 3))
def _run(xflat, comb, N, D):
  NBLK = N // NB
  KMAX = (NBLK + 31) // 32
  if KMAX % 2:
    KMAX += 1

  adj = pl.pallas_call(
      _adjust_body,
      out_shape=jax.ShapeDtypeStruct((1, FRAME), jnp.int32),
  )(xflat.reshape(-1, FRAME))

  mesh = plsc.VectorSubcoreMesh(core_axis_name="c", subcore_axis_name="s")
  body = functools.partial(_sc_body, NBLK, KMAX)
  DW = D // 2
  return pl.kernel(
      body,
      out_type=jax.ShapeDtypeStruct((N, DW), jnp.int32),
      mesh=mesh,
      compiler_params=pltpu.CompilerParams(needs_layout_passes=False,
                                           use_tc_tiling_on_sc=False),
      scratch_types=[
          pltpu.VMEM((WB,), jnp.int32),
          pltpu.VMEM((WB,), jnp.int32),
          pltpu.VMEM((1, FRAME), jnp.int32),
          pltpu.VMEM((GROUPS, GROW), jnp.int32),
          pltpu.VMEM((GROUPS, GROW), jnp.int32),
          pltpu.VMEM((WB, DW), jnp.int32),
          pltpu.VMEM((WB, DW), jnp.int32),
          pltpu.VMEM((NB, DW), jnp.int32),
          pltpu.VMEM((NB, DW), jnp.int32),
          pltpu.VMEM_SHARED((F * V, DW), jnp.int32),
          pltpu.SemaphoreType.DMA,
          pltpu.SemaphoreType.DMA,
          pltpu.SemaphoreType.DMA,
          pltpu.SemaphoreType.DMA,
          pltpu.SemaphoreType.DMA,
          pltpu.SemaphoreType.DMA,
      ],
  )(xflat, adj, comb)


def kernel(x, tables):
  N = x.shape[0]
  D = tables.shape[-1]
  xflat = x.reshape(-1)
  comb16 = tables.reshape(-1, D).astype(jnp.bfloat16)
  comb = jax.lax.bitcast_convert_type(
      comb16.reshape(-1, D // 2, 2), jnp.int32)
  out_i32 = _run(xflat, comb, N, D)
  out16 = jax.lax.bitcast_convert_type(out_i32, jnp.bfloat16)
  return out16.reshape(N, D).astype(jnp.float32)


# speedup vs baseline: 1.0203x; 1.0203x over previous
import functools

import jax
import jax.numpy as jnp
from jax import lax
from jax.experimental import pallas as pl
from jax.experimental.pallas import tpu as pltpu
from jax.experimental.pallas import tpu_sc as plsc

NC = 2
NS = 16
L = 16

F = 9
V = 1000

NB = 32
WB = NB * F
FRAME = F * L
VPB = WB // L
GROUPS = 3
GROW = WB // GROUPS
SPMEM_GROUPS = 2

INT_MAX = 2147483647


def _adjust_body(x_ref, adj_ref):
  m = jnp.min(x_ref[...], axis=0, keepdims=True)
  feat = lax.broadcasted_iota(jnp.int32, (1, FRAME), 1) % F
  adj = jnp.zeros((1, FRAME), jnp.int32)
  for fi in range(F):
    mask = feat == fi
    mfi = jnp.min(jnp.where(mask, m, INT_MAX), axis=1, keepdims=True)
    adj = jnp.where(mask, V * fi - mfi, adj)
  adj_ref[...] = adj


def _sc_body(NBLK, KMAX, xflat, adj_hbm, comb, out,
             xbuf0, xbuf1, adjbuf, idxbuf0, idxbuf1,
             gbuf0, gbuf1, obuf0, obuf1, stab,
             gsem0, gsem1, hsem0, hsem1, osem0, osem1):
  c = lax.axis_index("c")
  s = lax.axis_index("s")
  wid = s * NC + c

  xbufs = (xbuf0, xbuf1)
  idxbufs = (idxbuf0, idxbuf1)
  gbufs = (gbuf0, gbuf1)
  obufs = (obuf0, obuf1)
  gsems = (gsem0, gsem1)
  hsems = (hsem0, hsem1)
  osems = (osem0, osem1)

  @pl.when(s == 0)
  def _():
    pltpu.sync_copy(comb, stab)

  plsc.subcore_barrier()

  pltpu.sync_copy(adj_hbm, adjbuf)

  SROWS = SPMEM_GROUPS * GROW

  def prep(j, p):
    xbuf, idxbuf = xbufs[p], idxbufs[p]
    pltpu.sync_copy(xflat.at[pl.ds(j * WB, WB)], xbuf)
    for v in range(VPB):
      idx = xbuf[pl.ds(v * L, L)] + adjbuf[pl.ds((v % F) * L, L)]
      idxbuf[v // (GROW // L), pl.ds((v % (GROW // L)) * L, L)] = idx
    for g in range(GROUPS):
      if g < SPMEM_GROUPS:
        pltpu.async_copy(stab.at[idxbuf.at[g]],
                         gbufs[p].at[pl.ds(g * GROW, GROW)], gsems[p])
      else:
        pltpu.async_copy(comb.at[idxbuf.at[g]],
                         gbufs[p].at[pl.ds(g * GROW, GROW)], hsems[p])

  def wait_gathers(p):
    pltpu.make_async_copy(stab.at[pl.ds(0, SROWS)],
                          gbufs[p].at[pl.ds(0, SROWS)], gsems[p]).wait()
    pltpu.make_async_copy(comb.at[pl.ds(0, WB - SROWS)],
                          gbufs[p].at[pl.ds(SROWS, WB - SROWS)],
                          hsems[p]).wait()

  OW = NB * 64

  def accum_and_out(j, p):
    gbuf, obuf = gbufs[p], obufs[p]

    def node_body(n, _):
      for cc in range(4):
        acc = plsc.bitcast(gbuf[n * F, pl.ds(cc * L, L)], jnp.bfloat16)
        for fi in range(1, F):
          acc = acc + plsc.bitcast(gbuf[n * F + fi, pl.ds(cc * L, L)],
                                   jnp.bfloat16)
        obuf[pl.ds(n * 64 + cc * L, L)] = plsc.bitcast(acc, jnp.int32)
      return 0

    lax.fori_loop(0, NB, node_body, 0)
    pltpu.async_copy(obuf, out.at[pl.ds(j * OW, OW)], osems[p])

  def step(k, p):
    jn = wid + 32 * (k + 1)

    @pl.when(jn < NBLK)
    def _():
      prep(jn, 1 - p)

    jc = wid + 32 * k

    @pl.when(jc < NBLK)
    def _():
      wait_gathers(p)

      @pl.when(k >= 2)
      def _():
        pltpu.make_async_copy(obufs[p],
                              out.at[pl.ds((jc - 64) * NB * 64, NB * 64)],
                              osems[p]).wait()

      accum_and_out(jc, p)

  prep(wid, 0)

  def pair_body(m, _):
    step(2 * m, 0)
    step(2 * m + 1, 1)
    return 0

  lax.fori_loop(0, KMAX // 2, pair_body, 0)

  kstar = (NBLK - 1 - wid) // 32
  for p in (0, 1):
    kp_last = kstar - ((kstar - p) % 2)

    @pl.when(kp_last >= 0)
    def _():
      pltpu.make_async_copy(obufs[p],
                            out.at[pl.ds((wid + 32 * kp_last) * NB * 64,
                                         NB * 64)],
                            osems[p]).wait()


@functools.partial(jax.jit, static_argnums=(2, 3))
def _run(xflat, comb, N, D):
  NBLK = N // NB
  KMAX = (NBLK + 31) // 32
  if KMAX % 2:
    KMAX += 1

  adj = pl.pallas_call(
      _adjust_body,
      out_shape=jax.ShapeDtypeStruct((1, FRAME), jnp.int32),
  )(xflat.reshape(-1, FRAME)).reshape(FRAME)

  mesh = plsc.VectorSubcoreMesh(core_axis_name="c", subcore_axis_name="s")
  body = functools.partial(_sc_body, NBLK, KMAX)
  DW = D // 2
  return pl.kernel(
      body,
      out_type=jax.ShapeDtypeStruct((N * DW,), jnp.int32),
      mesh=mesh,
      compiler_params=pltpu.CompilerParams(needs_layout_passes=False,
                                           use_tc_tiling_on_sc=False),
      scratch_types=[
          pltpu.VMEM((WB,), jnp.int32),
          pltpu.VMEM((WB,), jnp.int32),
          pltpu.VMEM((FRAME,), jnp.int32),
          pltpu.VMEM((GROUPS, GROW), jnp.int32),
          pltpu.VMEM((GROUPS, GROW), jnp.int32),
          pltpu.VMEM((WB, DW), jnp.int32),
          pltpu.VMEM((WB, DW), jnp.int32),
          pltpu.VMEM((NB * DW,), jnp.int32),
          pltpu.VMEM((NB * DW,), jnp.int32),
          pltpu.VMEM_SHARED((F * V, DW), jnp.int32),
          pltpu.SemaphoreType.DMA,
          pltpu.SemaphoreType.DMA,
          pltpu.SemaphoreType.DMA,
          pltpu.SemaphoreType.DMA,
          pltpu.SemaphoreType.DMA,
          pltpu.SemaphoreType.DMA,
      ],
  )(xflat, adj, comb)


def kernel(x, tables):
  N = x.shape[0]
  D = tables.shape[-1]
  xflat = x.reshape(-1)
  comb16 = tables.reshape(-1, D).astype(jnp.bfloat16)
  comb = jax.lax.bitcast_convert_type(
      comb16.reshape(-1, D // 2, 2), jnp.int32)
  out_i32 = _run(xflat, comb, N, D)
  out16 = jax.lax.bitcast_convert_type(out_i32, jnp.bfloat16)
  return out16.reshape(N, D).astype(jnp.float32)


# speedup vs baseline: 1.4460x; 1.4172x over previous
import functools

import jax
import jax.numpy as jnp
import numpy as np
from jax import lax
from jax.experimental import pallas as pl
from jax.experimental.pallas import tpu as pltpu
from jax.experimental.pallas import tpu_sc as plsc

NC = 2
NS = 16
L = 16

F = 9
V = 1000

NB = 32
WB = NB * F
FRAME = F * L
VPB = WB // L
GROUPS = 3
GROW = WB // GROUPS

INT_MAX = 2147483647

_FEAT = np.arange(FRAME, dtype=np.int32) % F


def _adjust_body(x_ref, adj_ref):
  m = jnp.min(x_ref[...], axis=0, keepdims=True)
  feat = lax.broadcasted_iota(jnp.int32, (1, FRAME), 1) % F
  adj = jnp.zeros((1, FRAME), jnp.int32)
  for fi in range(F):
    mask = feat == fi
    mfi = jnp.min(jnp.where(mask, m, INT_MAX), axis=1, keepdims=True)
    adj = jnp.where(mask, V * fi - mfi, adj)
  adj_ref[...] = adj


def _sc_body(NBLK, KMAX, xflat, adj_hbm, comb, out,
             xbuf0, xbuf1, adjbuf, idxbuf0, idxbuf1, gbuf0, gbuf1,
             obuf0, obuf1, gsem0, gsem1, osem0, osem1, xsem0, xsem1):
  c = lax.axis_index("c")
  s = lax.axis_index("s")
  wid = s * NC + c

  xbufs = (xbuf0, xbuf1)
  idxbufs = (idxbuf0, idxbuf1)
  gbufs = (gbuf0, gbuf1)
  obufs = (obuf0, obuf1)
  gsems = (gsem0, gsem1)
  osems = (osem0, osem1)
  xsems = (xsem0, xsem1)

  pltpu.sync_copy(adj_hbm, adjbuf)

  def fire_x(j, p):
    pltpu.async_copy(xflat.at[pl.ds(j * WB, WB)], xbufs[p], xsems[p])

  def prep(j, p):
    xbuf, idxbuf = xbufs[p], idxbufs[p]
    pltpu.make_async_copy(xflat.at[pl.ds(0, WB)], xbuf, xsems[p]).wait()
    for v in range(VPB):
      idx = xbuf[pl.ds(v * L, L)] + adjbuf[0, pl.ds((v % F) * L, L)]
      idxbuf[v // (GROW // L), pl.ds((v % (GROW // L)) * L, L)] = idx
    for g in range(GROUPS):
      pltpu.async_copy(comb.at[idxbuf.at[g]],
                       gbufs[p].at[pl.ds(g * GROW, GROW)], gsems[p])

  def wait_gathers(p):
    pltpu.make_async_copy(comb.at[pl.ds(0, WB)], gbufs[p], gsems[p]).wait()

  def accum_and_out(j, p):
    gbuf, obuf = gbufs[p], obufs[p]

    def node_body(n, _):
      for cc in range(8):
        acc = gbuf[n * F, pl.ds(cc * L, L)]
        for fi in range(1, F):
          acc = acc + gbuf[n * F + fi, pl.ds(cc * L, L)]
        obuf[n, pl.ds(cc * L, L)] = acc
      return 0

    lax.fori_loop(0, NB, node_body, 0)
    pltpu.async_copy(obuf, out.at[pl.ds(j * NB, NB)], osems[p])

  def step(k, p):
    jn2 = wid + 32 * (k + 2)

    @pl.when(jn2 < NBLK)
    def _():
      fire_x(jn2, p)

    jn = wid + 32 * (k + 1)

    @pl.when(jn < NBLK)
    def _():
      prep(jn, 1 - p)

    jc = wid + 32 * k

    @pl.when(jc < NBLK)
    def _():
      wait_gathers(p)

      @pl.when(k >= 2)
      def _():
        pltpu.make_async_copy(obufs[p],
                              out.at[pl.ds((jc - 64) * NB, NB)],
                              osems[p]).wait()

      accum_and_out(jc, p)

  fire_x(wid, 0)
  jn1 = wid + 32

  @pl.when(jn1 < NBLK)
  def _():
    fire_x(jn1, 1)

  prep(wid, 0)

  def pair_body(m, _):
    step(2 * m, 0)
    step(2 * m + 1, 1)
    return 0

  lax.fori_loop(0, KMAX // 2, pair_body, 0)

  kstar = (NBLK - 1 - wid) // 32
  for p in (0, 1):
    kp_last = kstar - ((kstar - p) % 2)

    @pl.when(kp_last >= 0)
    def _():
      pltpu.make_async_copy(obufs[p],
                            out.at[pl.ds((wid + 32 * kp_last) * NB, NB)],
                            osems[p]).wait()


@functools.partial(jax.jit, static_argnums=(2, 3))
def _run(xflat, comb, N, D):
  NBLK = N // NB
  KMAX = (NBLK + 31) // 32

  adj = pl.pallas_call(
      _adjust_body,
      out_shape=jax.ShapeDtypeStruct((1, FRAME), jnp.int32),
  )(xflat.reshape(-1, FRAME))

  mesh = plsc.VectorSubcoreMesh(core_axis_name="c", subcore_axis_name="s")
  body = functools.partial(_sc_body, NBLK, KMAX)
  return pl.kernel(
      body,
      out_type=jax.ShapeDtypeStruct((N, D), jnp.float32),
      mesh=mesh,
      scratch_types=[
          pltpu.VMEM((WB,), jnp.int32),
          pltpu.VMEM((WB,), jnp.int32),
          pltpu.VMEM((1, FRAME), jnp.int32),
          pltpu.VMEM((GROUPS, GROW), jnp.int32),
          pltpu.VMEM((GROUPS, GROW), jnp.int32),
          pltpu.VMEM((WB, 128), jnp.float32),
          pltpu.VMEM((WB, 128), jnp.float32),
          pltpu.VMEM((NB, 128), jnp.float32),
          pltpu.VMEM((NB, 128), jnp.float32),
          pltpu.SemaphoreType.DMA,
          pltpu.SemaphoreType.DMA,
          pltpu.SemaphoreType.DMA,
          pltpu.SemaphoreType.DMA,
          pltpu.SemaphoreType.DMA,
          pltpu.SemaphoreType.DMA,
      ],
  )(xflat, adj, comb)


def kernel(x, tables):
  N = x.shape[0]
  D = tables.shape[-1]
  xflat = x.reshape(-1)
  comb = tables.reshape(-1, D)
  return _run(xflat, comb, N, D)
